# baseline (speedup 1.0000x reference)
import jax
import jax.numpy as jnp
from jax import lax
from jax.experimental import pallas as pl
from jax.experimental.pallas import tpu as pltpu
from jax.experimental.pallas import tpu_sc as plsc

DIM = 4096
NW = 32
CHUNK = 8
NCHUNKS = 100
MAX_PER_W = 4


def _gather_body(idx_hbm, table_hbm, out_hbm, idx_v, rows_v, sem):
    wid = lax.axis_index("s") * 2 + lax.axis_index("c")
    for k in range(MAX_PER_W):
        c = wid + NW * k

        @pl.when(c < NCHUNKS)
        def _():
            base = c * CHUNK
            pltpu.sync_copy(idx_hbm.at[pl.ds(base, CHUNK)], idx_v)
            pltpu.async_copy(table_hbm.at[idx_v], rows_v, sem).wait()
            pltpu.sync_copy(rows_v, out_hbm.at[pl.ds(base, CHUNK)])


@jax.jit
def kernel(indices, embedding_table):
    b, t = indices.shape
    n = b * t
    idx_flat = indices.reshape(n).astype(jnp.int32)
    mesh = plsc.VectorSubcoreMesh(core_axis_name="c", subcore_axis_name="s")
    out = pl.kernel(
        _gather_body,
        mesh=mesh,
        out_type=jax.ShapeDtypeStruct((n, DIM), jnp.float32),
        scratch_types=[
            pltpu.VMEM((CHUNK,), jnp.int32),
            pltpu.VMEM((CHUNK, DIM), jnp.float32),
            pltpu.SemaphoreType.DMA,
        ],
    )(idx_flat, embedding_table)
    return out.reshape(b, t, DIM)


# speedup vs baseline: 1.1195x; 1.1195x over previous
import jax
import jax.numpy as jnp
from jax import lax
from jax.experimental import pallas as pl
from jax.experimental.pallas import tpu as pltpu
from jax.experimental.pallas import tpu_sc as plsc

DIM = 4096
NW = 32
CHUNK = 8
NCHUNKS = 100
IDX_LOAD = 32


def _gather_body(idx_hbm, table_hbm, out_hbm, idx_v, rows0, rows1,
                 g0, g1, w0, w1):
    wid = lax.axis_index("s") * 2 + lax.axis_index("c")
    start = 3 * wid + jnp.minimum(wid, 4)
    rows = (rows0, rows1)
    gsem = (g0, g1)
    wsem = (w0, w1)

    pltpu.sync_copy(idx_hbm.at[pl.ds(start * CHUNK, IDX_LOAD)], idx_v)

    def gather(k):
        return pltpu.make_async_copy(
            table_hbm.at[idx_v.at[pl.ds(k * CHUNK, CHUNK)]],
            rows[k % 2], gsem[k % 2])

    def write(k):
        return pltpu.make_async_copy(
            rows[k % 2], out_hbm.at[pl.ds((start + k) * CHUNK, CHUNK)],
            wsem[k % 2])

    gather(0).start()
    gather(1).start()

    gather(0).wait()
    write(0).start()
    write(0).wait()
    gather(2).start()

    gather(1).wait()
    write(1).start()
    write(1).wait()

    @pl.when(wid < 4)
    def _():
        gather(3).start()

    gather(2).wait()
    write(2).start()

    @pl.when(wid < 4)
    def _():
        gather(3).wait()
        write(3).start()

    write(2).wait()

    @pl.when(wid < 4)
    def _():
        write(3).wait()


@jax.jit
def kernel(indices, embedding_table):
    b, t = indices.shape
    n = b * t
    idx_flat = indices.reshape(n).astype(jnp.int32)
    idx_flat = jnp.pad(idx_flat, (0, NW * IDX_LOAD - n))
    mesh = plsc.VectorSubcoreMesh(core_axis_name="c", subcore_axis_name="s")
    out = pl.kernel(
        _gather_body,
        mesh=mesh,
        out_type=jax.ShapeDtypeStruct((n, DIM), jnp.float32),
        scratch_types=[
            pltpu.VMEM((IDX_LOAD,), jnp.int32),
            pltpu.VMEM((CHUNK, DIM), jnp.float32),
            pltpu.VMEM((CHUNK, DIM), jnp.float32),
            pltpu.SemaphoreType.DMA,
            pltpu.SemaphoreType.DMA,
            pltpu.SemaphoreType.DMA,
            pltpu.SemaphoreType.DMA,
        ],
    )(idx_flat, embedding_table)
    return out.reshape(b, t, DIM)
